# baseline (speedup 1.0000x reference)
import functools

import jax
import jax.numpy as jnp
from jax import lax
from jax.experimental import pallas as pl
from jax.experimental.pallas import tpu as pltpu
from jax.experimental.pallas import tpu_sc as plsc

N = 100000
D = 128
G = 64
NC = 2
NS = 16
NW = NC * NS
L = 16
NJ = D // L

RPW = N // NW
WIN = 3136
CHUNK = 224
NCHUNK = WIN // CHUNK
GRP = 32
NGRP = CHUNK // GRP
IDS_PAD = WIN + 16

_mesh = plsc.VectorSubcoreMesh(core_axis_name="c", subcore_axis_name="s")
_params = pltpu.CompilerParams(use_tc_tiling_on_sc=False,
                               needs_layout_passes=False)

NEG_INF = float("-inf")


def _lane(vec, r):
  return jnp.max(jnp.where(lax.iota(jnp.int32, L) == r, vec, 0))


def _neg_inf_vec():
  return jnp.full((L,), NEG_INF, jnp.float32)


@functools.partial(
    pl.kernel,
    out_type=jax.ShapeDtypeStruct((NW, G, D), jnp.float32),
    mesh=_mesh,
    scratch_types=[
        pltpu.VMEM((IDS_PAD,), jnp.int32),
        pltpu.VMEM((2, CHUNK, D), jnp.float32),
        pltpu.VMEM((G, D), jnp.float32),
        pltpu.SemaphoreType.DMA,
        pltpu.SemaphoreType.DMA,
    ],
    compiler_params=_params,
)
def _partials(x_hbm, ids_hbm, part_hbm, ids_v, xbuf, acc, sem0, sem1):
  wid = lax.axis_index("s") * NC + lax.axis_index("c")
  row0 = wid * RPW
  start8 = jnp.minimum((row0 // 8) * 8, N - WIN)

  ids_dma = pltpu.async_copy(
      ids_hbm.at[pl.ds(start8, WIN)], ids_v.at[pl.ds(0, WIN)], sem0)

  def init_g(g, _):
    for j in range(NJ):
      acc[g, pl.ds(j * L, L)] = _neg_inf_vec()
    return 0
  lax.fori_loop(0, G, init_g, 0)

  pltpu.async_copy(x_hbm.at[pl.ds(start8, CHUNK)], xbuf.at[0], sem1)
  ids_dma.wait()

  sems = (sem1, sem0)

  def do_chunk(c, buf_idx, carry):
    @pl.when(c + 1 < NCHUNK)
    def _():
      pltpu.async_copy(x_hbm.at[pl.ds(start8 + (c + 1) * CHUNK, CHUNK)],
                       xbuf.at[1 - buf_idx], sems[1 - buf_idx])

    def do_group(g, carry):
      accv = list(carry[:NJ])
      cur_seg = carry[NJ]
      base = (c * NGRP + g) * GRP
      segs = jnp.full((L,), cur_seg, jnp.int32)
      vecs = [ids_v[pl.ds(base + v * L, L)] for v in range(GRP // L)]
      same = jnp.all(jnp.equal(vecs[0], segs))
      for v in vecs[1:]:
        same = jnp.logical_and(same, jnp.all(jnp.equal(v, segs)))

      def fast(accv, cur_seg):
        out = list(accv)
        for r in range(GRP):
          for j in range(NJ):
            out[j] = jnp.maximum(out[j], xbuf[buf_idx, g * GRP + r,
                                              pl.ds(j * L, L)])
        return tuple(out) + (cur_seg,)

      def slow(accv, cur_seg):
        for j in range(NJ):
          sl = pl.ds(j * L, L)
          acc[cur_seg, sl] = jnp.maximum(acc[cur_seg, sl], accv[j])
        for v in range(GRP // L):
          for r in range(L):
            seg = _lane(vecs[v], r)
            for j in range(NJ):
              sl = pl.ds(j * L, L)
              acc[seg, sl] = jnp.maximum(acc[seg, sl],
                                         xbuf[buf_idx, g * GRP + v * L + r, sl])
        return (tuple(_neg_inf_vec() for _ in range(NJ))
                + (_lane(vecs[-1], L - 1),))

      return lax.cond(same, fast, slow, accv, cur_seg)

    return lax.fori_loop(0, NGRP, do_group, carry)

  carry = tuple(_neg_inf_vec() for _ in range(NJ)) + (jnp.int32(0),)

  def chunk_pair(p, carry):
    for b in range(2):
      pltpu.make_async_copy(x_hbm.at[pl.ds(0, CHUNK)], xbuf.at[b],
                            sems[b]).wait()
      carry = do_chunk(p * 2 + b, b, carry)
    return carry

  carry = lax.fori_loop(0, NCHUNK // 2, chunk_pair, carry)

  accv = carry[:NJ]
  cur_seg = carry[NJ]
  for j in range(NJ):
    sl = pl.ds(j * L, L)
    acc[cur_seg, sl] = jnp.maximum(acc[cur_seg, sl], accv[j])

  pltpu.sync_copy(acc, part_hbm.at[wid])


@functools.partial(
    pl.kernel,
    out_type=jax.ShapeDtypeStruct((G, D), jnp.float32),
    mesh=_mesh,
    scratch_types=[
        pltpu.VMEM((NW, 2, D), jnp.float32),
        pltpu.VMEM((2, D), jnp.float32),
        pltpu.SemaphoreType.DMA,
    ],
    compiler_params=_params,
)
def _merge(part_hbm, out_hbm, buf, out_v, sem):
  wid = lax.axis_index("s") * NC + lax.axis_index("c")
  g0 = wid * 2
  descs = [pltpu.async_copy(part_hbm.at[p, pl.ds(g0, 2)], buf.at[p], sem)
           for p in range(NW)]
  for d_ in descs:
    d_.wait()
  for rr in range(2):
    for j in range(NJ):
      sl = pl.ds(j * L, L)
      m = buf[0, rr, sl]
      for p in range(1, NW):
        m = jnp.maximum(m, buf[p, rr, sl])
      out_v[rr, sl] = m
  pltpu.sync_copy(out_v, out_hbm.at[pl.ds(g0, 2)])


def kernel(x, batch):
  ids = batch.astype(jnp.int32)
  part = _partials(x, ids)
  return _merge(part)


# speedup vs baseline: 1.2973x; 1.2973x over previous
import functools

import jax
import jax.numpy as jnp
from jax import lax
from jax.experimental import pallas as pl
from jax.experimental.pallas import tpu as pltpu
from jax.experimental.pallas import tpu_sc as plsc

N = 100000
D = 128
G = 64
NC = 2
NS = 16
NW = NC * NS
L = 16
NJ = D // L

RPW = N // NW
WIN = 3136
CHUNK = 224
NCHUNK = WIN // CHUNK
GRP = 16
NGRP = CHUNK // GRP
IDS_PAD = WIN + 16

_mesh = plsc.VectorSubcoreMesh(core_axis_name="c", subcore_axis_name="s")
_params = pltpu.CompilerParams(use_tc_tiling_on_sc=False,
                               needs_layout_passes=False)

NEG_INF = float("-inf")


def _lane(vec, r):
  return jnp.max(jnp.where(lax.iota(jnp.int32, L) == r, vec, 0))


def _neg_inf_vec():
  return jnp.full((L,), NEG_INF, jnp.float32)


@functools.partial(
    pl.kernel,
    out_type=jax.ShapeDtypeStruct((NW, G, D), jnp.float32),
    mesh=_mesh,
    scratch_types=[
        pltpu.VMEM((IDS_PAD,), jnp.int32),
        pltpu.VMEM((2, CHUNK, D), jnp.float32),
        pltpu.VMEM((G, D), jnp.float32),
        pltpu.SemaphoreType.DMA,
        pltpu.SemaphoreType.DMA,
    ],
    compiler_params=_params,
)
def _partials(x_hbm, ids_hbm, part_hbm, ids_v, xbuf, acc, sem0, sem1):
  wid = lax.axis_index("s") * NC + lax.axis_index("c")
  row0 = wid * RPW
  start8 = jnp.minimum((row0 // 8) * 8, N - WIN)

  ids_dma = pltpu.async_copy(
      ids_hbm.at[pl.ds(start8, WIN)], ids_v.at[pl.ds(0, WIN)], sem0)

  def init_g(g, _):
    for j in range(NJ):
      acc[g, pl.ds(j * L, L)] = _neg_inf_vec()
    return 0
  lax.fori_loop(0, G, init_g, 0)

  pltpu.async_copy(x_hbm.at[pl.ds(start8, CHUNK)], xbuf.at[0], sem1)
  ids_dma.wait()

  sems = (sem1, sem0)

  def do_chunk(c, buf_idx, carry):
    @pl.when(c + 1 < NCHUNK)
    def _():
      pltpu.async_copy(x_hbm.at[pl.ds(start8 + (c + 1) * CHUNK, CHUNK)],
                       xbuf.at[1 - buf_idx], sems[1 - buf_idx])

    def do_group(g, carry):
      accv = list(carry[:NJ])
      cur_seg = carry[NJ]
      base = (c * NGRP + g) * GRP
      segs = jnp.full((L,), cur_seg, jnp.int32)
      vecs = [ids_v[pl.ds(base + v * L, L)] for v in range(GRP // L)]
      same = jnp.all(jnp.equal(vecs[0], segs))
      for v in vecs[1:]:
        same = jnp.logical_and(same, jnp.all(jnp.equal(v, segs)))

      def fast(accv, cur_seg):
        out = list(accv)
        for r in range(GRP):
          for j in range(NJ):
            out[j] = jnp.maximum(out[j], xbuf[buf_idx, g * GRP + r,
                                              pl.ds(j * L, L)])
        return tuple(out) + (cur_seg,)

      def slow(accv, cur_seg):
        for j in range(NJ):
          sl = pl.ds(j * L, L)
          acc[cur_seg, sl] = jnp.maximum(acc[cur_seg, sl], accv[j])
        for v in range(GRP // L):
          for r in range(L):
            seg = _lane(vecs[v], r)
            for j in range(NJ):
              sl = pl.ds(j * L, L)
              acc[seg, sl] = jnp.maximum(acc[seg, sl],
                                         xbuf[buf_idx, g * GRP + v * L + r, sl])
        return (tuple(_neg_inf_vec() for _ in range(NJ))
                + (_lane(vecs[-1], L - 1),))

      return lax.cond(same, fast, slow, accv, cur_seg)

    return lax.fori_loop(0, NGRP, do_group, carry)

  carry = tuple(_neg_inf_vec() for _ in range(NJ)) + (jnp.int32(0),)

  def chunk_pair(p, carry):
    for b in range(2):
      pltpu.make_async_copy(x_hbm.at[pl.ds(0, CHUNK)], xbuf.at[b],
                            sems[b]).wait()
      carry = do_chunk(p * 2 + b, b, carry)
    return carry

  carry = lax.fori_loop(0, NCHUNK // 2, chunk_pair, carry)

  accv = carry[:NJ]
  cur_seg = carry[NJ]
  for j in range(NJ):
    sl = pl.ds(j * L, L)
    acc[cur_seg, sl] = jnp.maximum(acc[cur_seg, sl], accv[j])

  pltpu.sync_copy(acc, part_hbm.at[wid])


@functools.partial(
    pl.kernel,
    out_type=jax.ShapeDtypeStruct((G, D), jnp.float32),
    mesh=_mesh,
    scratch_types=[
        pltpu.VMEM((NW, 2, D), jnp.float32),
        pltpu.VMEM((2, D), jnp.float32),
        pltpu.SemaphoreType.DMA,
    ],
    compiler_params=_params,
)
def _merge(part_hbm, out_hbm, buf, out_v, sem):
  wid = lax.axis_index("s") * NC + lax.axis_index("c")
  g0 = wid * 2
  descs = [pltpu.async_copy(part_hbm.at[p, pl.ds(g0, 2)], buf.at[p], sem)
           for p in range(NW)]
  for d_ in descs:
    d_.wait()
  for rr in range(2):
    for j in range(NJ):
      sl = pl.ds(j * L, L)
      m = buf[0, rr, sl]
      for p in range(1, NW):
        m = jnp.maximum(m, buf[p, rr, sl])
      out_v[rr, sl] = m
  pltpu.sync_copy(out_v, out_hbm.at[pl.ds(g0, 2)])


def kernel(x, batch):
  ids = batch.astype(jnp.int32)
  part = _partials(x, ids)
  return _merge(part)


# speedup vs baseline: 1.4435x; 1.1127x over previous
import functools

import jax
import jax.numpy as jnp
from jax import lax
from jax.experimental import pallas as pl
from jax.experimental.pallas import tpu as pltpu
from jax.experimental.pallas import tpu_sc as plsc

N = 100000
D = 128
G = 64
NC = 2
NS = 16
NW = NC * NS
L = 16
NJ = D // L

RPW = N // NW
WIN = 3136
CHUNK = 224
NCHUNK = WIN // CHUNK
GRP = 16
NGRP = CHUNK // GRP
IDS_PAD = WIN + 16

_mesh = plsc.VectorSubcoreMesh(core_axis_name="c", subcore_axis_name="s")
_params = pltpu.CompilerParams(use_tc_tiling_on_sc=False,
                               needs_layout_passes=False)

NEG_INF = float("-inf")


def _lane(vec, r):
  return jnp.max(jnp.where(lax.iota(jnp.int32, L) == r, vec, 0))


def _neg_inf_vec():
  return jnp.full((L,), NEG_INF, jnp.float32)


@functools.partial(
    pl.kernel,
    out_type=jax.ShapeDtypeStruct((NW, G, D), jnp.float32),
    mesh=_mesh,
    scratch_types=[
        pltpu.VMEM((IDS_PAD,), jnp.int32),
        pltpu.VMEM((2, CHUNK, D), jnp.float32),
        pltpu.VMEM((G, D), jnp.float32),
        pltpu.SemaphoreType.DMA,
        pltpu.SemaphoreType.DMA,
    ],
    compiler_params=_params,
)
def _partials(x_hbm, ids_hbm, part_hbm, ids_v, xbuf, acc, sem0, sem1):
  wid = lax.axis_index("s") * NC + lax.axis_index("c")
  row0 = wid * RPW
  start8 = jnp.minimum((row0 // 8) * 8, N - WIN)

  ids_dma = pltpu.async_copy(
      ids_hbm.at[pl.ds(start8, WIN)], ids_v.at[pl.ds(0, WIN)], sem0)

  def init_g(g, _):
    for j in range(NJ):
      acc[g, pl.ds(j * L, L)] = _neg_inf_vec()
    return 0
  lax.fori_loop(0, G, init_g, 0)

  pltpu.async_copy(x_hbm.at[pl.ds(start8, CHUNK)], xbuf.at[0], sem1)
  ids_dma.wait()

  sems = (sem1, sem0)

  def do_chunk(c, buf_idx, carry):
    @pl.when(c + 1 < NCHUNK)
    def _():
      pltpu.async_copy(x_hbm.at[pl.ds(start8 + (c + 1) * CHUNK, CHUNK)],
                       xbuf.at[1 - buf_idx], sems[1 - buf_idx])

    def do_group(g, carry):
      accv = list(carry[:NJ])
      cur_seg = carry[NJ]
      base = (c * NGRP + g) * GRP
      segs = jnp.full((L,), cur_seg, jnp.int32)
      vecs = [ids_v[pl.ds(base + v * L, L)] for v in range(GRP // L)]
      same = jnp.all(jnp.equal(vecs[0], segs))
      for v in vecs[1:]:
        same = jnp.logical_and(same, jnp.all(jnp.equal(v, segs)))

      def fast(accv, cur_seg):
        out = list(accv)
        for r in range(GRP):
          for j in range(NJ):
            out[j] = jnp.maximum(out[j], xbuf[buf_idx, g * GRP + r,
                                              pl.ds(j * L, L)])
        return tuple(out) + (cur_seg,)

      def slow(accv, cur_seg):
        for j in range(NJ):
          sl = pl.ds(j * L, L)
          acc[cur_seg, sl] = jnp.maximum(acc[cur_seg, sl], accv[j])
        def rmw_row(r, _):
          seg = _lane(vecs[0], r)
          for j in range(NJ):
            sl = pl.ds(j * L, L)
            acc[seg, sl] = jnp.maximum(acc[seg, sl],
                                       xbuf[buf_idx, g * GRP + r, sl])
          return 0
        lax.fori_loop(0, GRP, rmw_row, 0)
        return (tuple(_neg_inf_vec() for _ in range(NJ))
                + (_lane(vecs[-1], L - 1),))

      return lax.cond(same, fast, slow, accv, cur_seg)

    return lax.fori_loop(0, NGRP, do_group, carry)

  carry = tuple(_neg_inf_vec() for _ in range(NJ)) + (jnp.int32(0),)

  def chunk_pair(p, carry):
    for b in range(2):
      pltpu.make_async_copy(x_hbm.at[pl.ds(0, CHUNK)], xbuf.at[b],
                            sems[b]).wait()
      carry = do_chunk(p * 2 + b, b, carry)
    return carry

  carry = lax.fori_loop(0, NCHUNK // 2, chunk_pair, carry)

  accv = carry[:NJ]
  cur_seg = carry[NJ]
  for j in range(NJ):
    sl = pl.ds(j * L, L)
    acc[cur_seg, sl] = jnp.maximum(acc[cur_seg, sl], accv[j])

  pltpu.sync_copy(acc, part_hbm.at[wid])


def _merge_body(part_ref, out_ref):
  out_ref[...] = jnp.max(part_ref[...], axis=0)


_merge = pl.pallas_call(
    _merge_body,
    out_shape=jax.ShapeDtypeStruct((G, D), jnp.float32),
)


def kernel(x, batch):
  ids = batch.astype(jnp.int32)
  part = _partials(x, ids)
  return _merge(part)
